# baseline (speedup 1.0000x reference)
import functools

import jax
import jax.numpy as jnp
from jax import lax
from jax.experimental import pallas as pl
from jax.experimental.pallas import tpu as pltpu
from jax.experimental.pallas import tpu_sc as plsc

_VOCAB = 100000
_D = 128
_BATCH = 4096
_SEQ = 200
_N = _BATCH * _SEQ
_NC = 2
_NS = 16
_NW = _NC * _NS
_PER_W = _N // _NW
_CHUNK = 160
_NBUF = 4
_NSTEP = _PER_W // _CHUNK
assert _NSTEP % _NBUF == 0

_mesh = plsc.VectorSubcoreMesh(core_axis_name="c", subcore_axis_name="s")


@functools.partial(
    pl.kernel,
    mesh=_mesh,
    out_type=jax.ShapeDtypeStruct((_N, _D), jnp.float32),
    scratch_types=[
        pltpu.VMEM((_PER_W,), jnp.int32),
        pltpu.VMEM((_NBUF, _CHUNK, _D), jnp.float32),
    ] + [pltpu.SemaphoreType.DMA] * (2 * _NBUF),
)
def _gather(idx_hbm, table_hbm, out_hbm, idx_v, rows_v, *sems):
    wid = lax.axis_index("s") * _NC + lax.axis_index("c")
    base = wid * _PER_W
    sg = list(sems[:_NBUF])
    ss = list(sems[_NBUF:])

    pltpu.async_copy(idx_hbm.at[pl.ds(base, _PER_W)], idx_v, sg[0]).wait()

    def start_gather(k, b):
        pltpu.async_copy(
            table_hbm.at[idx_v.at[pl.ds(k * _CHUNK, _CHUNK)]],
            rows_v.at[b], sg[b])

    def start_scatter(k, b):
        pltpu.async_copy(
            rows_v.at[b], out_hbm.at[pl.ds(base + k * _CHUNK, _CHUNK)], ss[b])

    def wait(sem):
        pltpu.make_async_copy(
            out_hbm.at[pl.ds(0, _CHUNK)], rows_v.at[0], sem).wait()

    for b in range(_NBUF - 1):
        start_gather(b, b)

    def body(j, carry):
        for i in range(_NBUF):
            b = i
            bn = (i + _NBUF - 1) % _NBUF
            k = j * _NBUF + i
            kn = k + _NBUF - 1
            @pl.when(kn < _NSTEP)
            def _():
                @pl.when(k >= 1)
                def _():
                    wait(ss[bn])
                start_gather(kn, bn)

            wait(sg[b])
            start_scatter(k, b)
        return carry

    lax.fori_loop(0, _NSTEP // _NBUF, body, 0)

    for b in range(_NBUF):
        wait(ss[b])


def kernel(input_ids, token_embedding_weight, positional_embedding_weight):
    del positional_embedding_weight
    flat = input_ids.reshape(_N)
    out = _gather(flat, token_embedding_weight)
    return out.reshape(_BATCH, _SEQ, _D)


# speedup vs baseline: 1.0001x; 1.0001x over previous
import functools

import jax
import jax.numpy as jnp
from jax import lax
from jax.experimental import pallas as pl
from jax.experimental.pallas import tpu as pltpu
from jax.experimental.pallas import tpu_sc as plsc

_VOCAB = 100000
_D = 128
_BATCH = 4096
_SEQ = 200
_N = _BATCH * _SEQ
_NC = 2
_NS = 16
_NW = _NC * _NS
_PER_W = _N // _NW
_CHUNK = 160
_NBUF = 4
_NSTEP = _PER_W // _CHUNK
assert _NSTEP % _NBUF == 0

_mesh = plsc.VectorSubcoreMesh(core_axis_name="c", subcore_axis_name="s")


@functools.partial(
    pl.kernel,
    mesh=_mesh,
    out_type=jax.ShapeDtypeStruct((_N, _D), jnp.float32),
    scratch_types=[
        pltpu.VMEM((_PER_W,), jnp.int32),
        pltpu.VMEM((_NBUF, _CHUNK, _D), jnp.float32),
    ] + [pltpu.SemaphoreType.DMA] * (2 * _NBUF),
)
def _gather(idx_hbm, table_hbm, out_hbm, idx_v, rows_v, *sems):
    wid = lax.axis_index("s") * _NC + lax.axis_index("c")
    base = wid * _PER_W
    sg = list(sems[:_NBUF])
    ss = list(sems[_NBUF:])

    pltpu.async_copy(idx_hbm.at[pl.ds(base, _PER_W)], idx_v, sg[0]).wait()

    def start_gather(k, b):
        pltpu.async_copy(
            table_hbm.at[idx_v.at[pl.ds(k * _CHUNK, _CHUNK)]],
            rows_v.at[b], sg[b])

    def start_scatter(k, b):
        pltpu.async_copy(
            rows_v.at[b], out_hbm.at[pl.ds(base + k * _CHUNK, _CHUNK)], ss[b])

    def wait(sem):
        pltpu.make_async_copy(
            out_hbm.at[pl.ds(0, _CHUNK)], rows_v.at[0], sem).wait()

    for b in range(_NBUF - 1):
        start_gather(b, b)

    def body(j, carry):
        for i in range(_NBUF):
            b = i
            bn = (i + _NBUF - 1) % _NBUF
            k = j * _NBUF + i
            kn = k + _NBUF - 1
            @pl.when(kn < _NSTEP)
            def _():
                @pl.when(k >= 1)
                def _():
                    wait(ss[bn])
                start_gather(kn, bn)

            wait(sg[b])
            start_scatter(k, b)
        return carry

    lax.fori_loop(0, _NSTEP // _NBUF, body, 0)

    for b in range(_NBUF):
        wait(ss[b])


def kernel(input_ids, token_embedding_weight, positional_embedding_weight):
    del positional_embedding_weight
    flat = input_ids.reshape(_N).astype(jnp.int32)
    out = _gather(flat, token_embedding_weight)
    return out.reshape(_BATCH, _SEQ, _D)
